# Initial kernel scaffold; baseline (speedup 1.0000x reference)
import jax
import jax.numpy as jnp
from jax.experimental import pallas as pl


def kernel(input_features, table):
    raise NotImplementedError("write your pallas kernel here")


# speedup vs baseline: 5.1690x; 5.1690x over previous
import functools

import jax
import jax.numpy as jnp
from jax import lax
from jax.experimental import pallas as pl
from jax.experimental.pallas import tpu as pltpu
from jax.experimental.pallas import tpu_sc as plsc

OUTPUT_DIM = 32
NUM_FIELDS = 6
FIELD_SIZE = 100
BATCH = 16384

NC, NS, L = 2, 16, 16
NW = NC * NS
B_PER_W = BATCH // NW
PER_W = B_PER_W * NUM_FIELDS
CHUNK = 128
NCHUNK = PER_W // CHUNK
BLKS = B_PER_W // CHUNK

_mesh = plsc.VectorSubcoreMesh(
    core_axis_name="c", subcore_axis_name="s", num_cores=NC, num_subcores=NS
)


@functools.partial(
    pl.kernel,
    out_type=jax.ShapeDtypeStruct((BATCH, NUM_FIELDS * OUTPUT_DIM), jnp.float32),
    mesh=_mesh,
    scratch_types=[
        pltpu.VMEM((NCHUNK, CHUNK), jnp.int32),
        pltpu.VMEM((PER_W,), jnp.int32),
        pltpu.VMEM((PER_W, OUTPUT_DIM), jnp.float32),
        pltpu.SemaphoreType.DMA,
        pltpu.SemaphoreType.DMA,
    ],
    compiler_params=pltpu.CompilerParams(
        use_tc_tiling_on_sc=False, needs_layout_passes=False
    ),
)
def _embed_gather(feat_hbm, table_hbm, out_hbm, fv, idx_v, rows_v, sem, semw):
    wid = lax.axis_index("s") * NC + lax.axis_index("c")
    b0 = wid * B_PER_W
    pltpu.sync_copy(feat_hbm.at[pl.ds(wid * NCHUNK, NCHUNK)], fv)

    lane = lax.iota(jnp.int32, L)
    six = jnp.full((L,), NUM_FIELDS, jnp.int32)

    def reorder(c, _):
        for k in range(CHUNK // L):
            pos = lane + (c * CHUNK + k * L)
            f_v = lax.rem(pos, six)
            b_v = lax.div(pos, six)
            v = fv[c, pl.ds(k * L, L)]
            plsc.store_scatter(idx_v, [f_v * B_PER_W + b_v], v + f_v * FIELD_SIZE)
        return 0

    lax.fori_loop(0, NCHUNK, reorder, 0)

    copies = [
        pltpu.async_copy(
            table_hbm.at[idx_v.at[pl.ds(s * CHUNK, CHUNK)]],
            rows_v.at[pl.ds(s * CHUNK, CHUNK)],
            sem,
        )
        for s in range(NCHUNK)
    ]
    for cp in copies:
        cp.wait()
    outs = [
        pltpu.async_copy(
            rows_v.at[pl.ds(f * B_PER_W, B_PER_W)],
            out_hbm.at[pl.ds(b0, B_PER_W), pl.ds(f * OUTPUT_DIM, OUTPUT_DIM)],
            semw,
        )
        for f in range(NUM_FIELDS)
    ]
    for cp in outs:
        cp.wait()


def kernel(input_features, table):
    feat = input_features.astype(jnp.int32).reshape(NW * NCHUNK, CHUNK)
    return _embed_gather(feat, table)
